# baseline (speedup 1.0000x reference)
import functools

import jax
import jax.numpy as jnp
import numpy as np
from jax import lax
from jax.experimental import pallas as pl
from jax.experimental.pallas import tpu as pltpu
from jax.experimental.pallas import tpu_sc as plsc

POOL = 7
NPTS = POOL * POOL
NPAD = 56
LEVEL_W = (128, 64, 32, 16)
LEVEL_OFF = (0, 16384, 20480, 21504)
TABLE_ROWS = 21760


def _conv2d(x, W, b, pad):
    out = jax.lax.conv_general_dilated(
        x, W, (1, 1), [(pad, pad), (pad, pad)],
        dimension_numbers=('NCHW', 'OIHW', 'NCHW'))
    return out + b[None, :, None, None]


def _upsample_add(x, y):
    B, C, H, W = y.shape
    return jax.image.resize(x, (x.shape[0], x.shape[1], H, W), method='bilinear') + y


def _roi_meta(rois, im_info):
    R = rois.shape[0]
    h = rois[:, 4] - rois[:, 2] + 1.0
    w = rois[:, 3] - rois[:, 1] + 1.0
    lvl = jnp.floor(jnp.log(jnp.sqrt(h * w) / 224.0) / np.log(2.0) + 4.0)
    lvl = jnp.clip(lvl, 2.0, 5.0)
    li = lvl.astype(jnp.int32) - 2
    Wf = jnp.array(LEVEL_W, jnp.float32)[li]
    Wi = jnp.array(LEVEL_W, jnp.int32)[li]
    off = jnp.array(LEVEL_OFF, jnp.int32)[li]
    scale = Wf / im_info[0, 0]
    x1 = rois[:, 1] * scale
    y1 = rois[:, 2] * scale
    x2 = rois[:, 3] * scale
    y2 = rois[:, 4] * scale
    bw = jnp.maximum(x2 - x1 + 1.0, 1.0) / POOL
    bh = jnp.maximum(y2 - y1 + 1.0, 1.0) / POOL
    jj = jnp.arange(POOL, dtype=jnp.float32) + 0.5
    xs = x1[:, None] + jj[None, :] * bw[:, None]
    ys = y1[:, None] + jj[None, :] * bh[:, None]
    x0f = jnp.floor(xs)
    y0f = jnp.floor(ys)
    lx = xs - x0f
    ly = ys - y0f
    wmax = Wi[:, None] - 1
    x0 = jnp.clip(x0f.astype(jnp.int32), 0, wmax)
    x1i = jnp.clip(x0 + 1, 0, wmax)
    y0 = jnp.clip(y0f.astype(jnp.int32), 0, wmax)
    y1i = jnp.clip(y0 + 1, 0, wmax)
    ry0 = off[:, None] + y0 * Wi[:, None]
    ry1 = off[:, None] + y1i * Wi[:, None]

    def mk(rowy, xcol):
        return (rowy[:, :, None] + xcol[:, None, :]).reshape(R, NPTS)

    def mw(a, b):
        return (a[:, :, None] * b[:, None, :]).reshape(R, NPTS)

    idx = jnp.stack([mk(ry0, x0), mk(ry0, x1i), mk(ry1, x0), mk(ry1, x1i)], 1)
    wts = jnp.stack([mw(1.0 - ly, 1.0 - lx), mw(1.0 - ly, lx),
                     mw(ly, 1.0 - lx), mw(ly, lx)], 1)
    pad = ((0, 0), (0, 0), (0, NPAD - NPTS))
    return jnp.pad(idx, pad).astype(jnp.int32), jnp.pad(wts, pad)


def _sc_gather(table, idx):
    R = idx.shape[0]
    rpw = R // 32
    mesh = plsc.VectorSubcoreMesh(core_axis_name="c", subcore_axis_name="s")

    @functools.partial(
        pl.kernel, mesh=mesh,
        out_type=jax.ShapeDtypeStruct((R, 4, NPAD, 256), jnp.float32),
        scratch_types=[
            pltpu.VMEM((rpw, 4, NPAD), jnp.int32),
            pltpu.VMEM((2, 4, NPAD, 256), jnp.float32),
            pltpu.SemaphoreType.DMA,
            pltpu.SemaphoreType.DMA,
            pltpu.SemaphoreType.DMA,
            pltpu.SemaphoreType.DMA,
        ],
    )
    def k(table_hbm, idx_hbm, out_hbm, idx_v, rows_v, gsem0, gsem1, ssem0, ssem1):
        wid = lax.axis_index("s") * 2 + lax.axis_index("c")
        base = wid * rpw
        pltpu.sync_copy(idx_hbm.at[pl.ds(base, rpw)], idx_v)
        gsem = (gsem0, gsem1)
        ssem = (ssem0, ssem1)
        pend_g = [None, None]
        pend_s = [None, None]

        def fire(i):
            s = i % 2
            pend_g[s] = [
                pltpu.async_copy(table_hbm.at[idx_v.at[i, kk]],
                                 rows_v.at[s, kk], gsem[s])
                for kk in range(4)
            ]

        fire(0)
        for i in range(rpw):
            s = i % 2
            if i + 1 < rpw:
                if pend_s[1 - s] is not None:
                    pend_s[1 - s].wait()
                fire(i + 1)
            for h in pend_g[s]:
                h.wait()
            pend_s[s] = pltpu.async_copy(rows_v.at[s], out_hbm.at[base + i],
                                         ssem[s])
        pend_s[0].wait()
        pend_s[1].wait()

    return k(table, idx)


def _combine_body(w_ref, g_ref, o_ref):
    acc = (g_ref[:, 0] * w_ref[:, 0] + g_ref[:, 1] * w_ref[:, 1]
           + g_ref[:, 2] * w_ref[:, 2] + g_ref[:, 3] * w_ref[:, 3])
    o_ref[...] = acc[:, :NPTS, :]


def _tc_combine(g, wts):
    R = g.shape[0]
    BR = 8
    return pl.pallas_call(
        _combine_body,
        grid=(R // BR,),
        in_specs=[pl.BlockSpec((BR, 4, NPAD, 1), lambda i: (i, 0, 0, 0)),
                  pl.BlockSpec((BR, 4, NPAD, 256), lambda i: (i, 0, 0, 0))],
        out_specs=pl.BlockSpec((BR, NPTS, 256), lambda i: (i, 0, 0)),
        out_shape=jax.ShapeDtypeStruct((R, NPTS, 256), jnp.float32),
    )(wts[..., None], g)


def kernel(c2, c3, c4, c5, rois, im_info, Wt, bt, Wl1, bl1, Wl2, bl2, Wl3, bl3,
           Ws1, bs1, Ws2, bs2, Ws3, bs3):
    p5 = _conv2d(c5, Wt, bt, 0)
    p4 = _conv2d(_upsample_add(p5, _conv2d(c4, Wl1, bl1, 0)), Ws1, bs1, 1)
    p3 = _conv2d(_upsample_add(p4, _conv2d(c3, Wl2, bl2, 0)), Ws2, bs2, 1)
    p2 = _conv2d(_upsample_add(p3, _conv2d(c2, Wl3, bl3, 0)), Ws3, bs3, 1)

    C = Wt.shape[0]
    table = jnp.concatenate(
        [jnp.transpose(p, (0, 2, 3, 1)).reshape(-1, C) for p in (p2, p3, p4, p5)], 0)

    R = rois.shape[0]
    idx, wts = _roi_meta(rois, im_info)
    g = _sc_gather(table, idx)
    out = _tc_combine(g, wts)
    return jnp.transpose(out.reshape(R, POOL, POOL, C), (0, 3, 1, 2))


# speedup vs baseline: 1.0002x; 1.0002x over previous
import functools

import jax
import jax.numpy as jnp
import numpy as np
from jax import lax
from jax.experimental import pallas as pl
from jax.experimental.pallas import tpu as pltpu
from jax.experimental.pallas import tpu_sc as plsc

POOL = 7
NPTS = POOL * POOL
NPAD = 56
LEVEL_W = (128, 64, 32, 16)
LEVEL_OFF = (0, 16384, 20480, 21504)
TABLE_ROWS = 21760


def _conv2d(x, W, b, pad):
    out = jax.lax.conv_general_dilated(
        x, W, (1, 1), [(pad, pad), (pad, pad)],
        dimension_numbers=('NCHW', 'OIHW', 'NCHW'))
    return out + b[None, :, None, None]


def _upsample_add(x, y):
    B, C, H, W = y.shape
    return jax.image.resize(x, (x.shape[0], x.shape[1], H, W), method='bilinear') + y


def _roi_meta(rois, im_info):
    R = rois.shape[0]
    h = rois[:, 4] - rois[:, 2] + 1.0
    w = rois[:, 3] - rois[:, 1] + 1.0
    lvl = jnp.floor(jnp.log(jnp.sqrt(h * w) / 224.0) / np.log(2.0) + 4.0)
    lvl = jnp.clip(lvl, 2.0, 5.0)
    li = lvl.astype(jnp.int32) - 2
    Wf = jnp.array(LEVEL_W, jnp.float32)[li]
    Wi = jnp.array(LEVEL_W, jnp.int32)[li]
    off = jnp.array(LEVEL_OFF, jnp.int32)[li]
    scale = Wf / im_info[0, 0]
    x1 = rois[:, 1] * scale
    y1 = rois[:, 2] * scale
    x2 = rois[:, 3] * scale
    y2 = rois[:, 4] * scale
    bw = jnp.maximum(x2 - x1 + 1.0, 1.0) / POOL
    bh = jnp.maximum(y2 - y1 + 1.0, 1.0) / POOL
    jj = jnp.arange(POOL, dtype=jnp.float32) + 0.5
    xs = x1[:, None] + jj[None, :] * bw[:, None]
    ys = y1[:, None] + jj[None, :] * bh[:, None]
    x0f = jnp.floor(xs)
    y0f = jnp.floor(ys)
    lx = xs - x0f
    ly = ys - y0f
    wmax = Wi[:, None] - 1
    x0 = jnp.clip(x0f.astype(jnp.int32), 0, wmax)
    x1i = jnp.clip(x0 + 1, 0, wmax)
    y0 = jnp.clip(y0f.astype(jnp.int32), 0, wmax)
    y1i = jnp.clip(y0 + 1, 0, wmax)
    ry0 = off[:, None] + y0 * Wi[:, None]
    ry1 = off[:, None] + y1i * Wi[:, None]

    def mk(rowy, xcol):
        return (rowy[:, :, None] + xcol[:, None, :]).reshape(R, NPTS)

    def mw(a, b):
        return (a[:, :, None] * b[:, None, :]).reshape(R, NPTS)

    idx = jnp.stack([mk(ry0, x0), mk(ry0, x1i), mk(ry1, x0), mk(ry1, x1i)], 1)
    wts = jnp.stack([mw(1.0 - ly, 1.0 - lx), mw(1.0 - ly, lx),
                     mw(ly, 1.0 - lx), mw(ly, lx)], 1)
    pad = ((0, 0), (0, 0), (0, NPAD - NPTS))
    return jnp.pad(idx, pad).astype(jnp.int32), jnp.pad(wts, pad)


def _sc_gather(table, idx):
    R = idx.shape[0]
    rpw = R // 32
    mesh = plsc.VectorSubcoreMesh(core_axis_name="c", subcore_axis_name="s")

    @functools.partial(
        pl.kernel, mesh=mesh,
        out_type=jax.ShapeDtypeStruct((R, 2, 2 * NPAD, 256), jnp.float32),
        scratch_types=[
            pltpu.VMEM((rpw, 2, 2 * NPAD), jnp.int32),
            pltpu.VMEM((2, 2, 2 * NPAD, 256), jnp.float32),
            pltpu.SemaphoreType.DMA,
            pltpu.SemaphoreType.DMA,
            pltpu.SemaphoreType.DMA,
            pltpu.SemaphoreType.DMA,
        ],
    )
    def k(table_hbm, idx_hbm, out_hbm, idx_v, rows_v, gsem0, gsem1, ssem0, ssem1):
        wid = lax.axis_index("s") * 2 + lax.axis_index("c")
        base = wid * rpw
        pltpu.sync_copy(idx_hbm.at[pl.ds(base, rpw)], idx_v)
        gsem = (gsem0, gsem1)
        ssem = (ssem0, ssem1)
        pend_g = [None, None]
        pend_s = [None, None]

        def fire(i):
            s = i % 2
            pend_g[s] = [
                pltpu.async_copy(table_hbm.at[idx_v.at[i, j]],
                                 rows_v.at[s, j], gsem[s])
                for j in range(2)
            ]

        fire(0)
        for i in range(rpw):
            s = i % 2
            if i + 1 < rpw:
                if pend_s[1 - s] is not None:
                    pend_s[1 - s].wait()
                fire(i + 1)
            for h in pend_g[s]:
                h.wait()
            pend_s[s] = pltpu.async_copy(rows_v.at[s], out_hbm.at[base + i],
                                         ssem[s])
        pend_s[0].wait()
        pend_s[1].wait()

    return k(table, idx)


def _combine_body(w_ref, g_ref, o_ref):
    acc = (g_ref[:, 0] * w_ref[:, 0] + g_ref[:, 1] * w_ref[:, 1]
           + g_ref[:, 2] * w_ref[:, 2] + g_ref[:, 3] * w_ref[:, 3])
    o_ref[...] = acc[:, :NPTS, :]


def _tc_combine(g, wts):
    R = g.shape[0]
    BR = 8
    return pl.pallas_call(
        _combine_body,
        grid=(R // BR,),
        in_specs=[pl.BlockSpec((BR, 4, NPAD, 1), lambda i: (i, 0, 0, 0)),
                  pl.BlockSpec((BR, 4, NPAD, 256), lambda i: (i, 0, 0, 0))],
        out_specs=pl.BlockSpec((BR, NPTS, 256), lambda i: (i, 0, 0)),
        out_shape=jax.ShapeDtypeStruct((R, NPTS, 256), jnp.float32),
    )(wts[..., None], g)


def kernel(c2, c3, c4, c5, rois, im_info, Wt, bt, Wl1, bl1, Wl2, bl2, Wl3, bl3,
           Ws1, bs1, Ws2, bs2, Ws3, bs3):
    p5 = _conv2d(c5, Wt, bt, 0)
    p4 = _conv2d(_upsample_add(p5, _conv2d(c4, Wl1, bl1, 0)), Ws1, bs1, 1)
    p3 = _conv2d(_upsample_add(p4, _conv2d(c3, Wl2, bl2, 0)), Ws2, bs2, 1)
    p2 = _conv2d(_upsample_add(p3, _conv2d(c2, Wl3, bl3, 0)), Ws3, bs3, 1)

    C = Wt.shape[0]
    table = jnp.concatenate(
        [jnp.transpose(p, (0, 2, 3, 1)).reshape(-1, C) for p in (p2, p3, p4, p5)], 0)

    R = rois.shape[0]
    idx, wts = _roi_meta(rois, im_info)
    g = _sc_gather(table, idx.reshape(R, 2, 2 * NPAD))
    out = _tc_combine(g.reshape(R, 4, NPAD, C), wts)
    return jnp.transpose(out.reshape(R, POOL, POOL, C), (0, 3, 1, 2))


# speedup vs baseline: 2.0510x; 2.0507x over previous
import functools

import jax
import jax.numpy as jnp
import numpy as np
from jax import lax
from jax.experimental import pallas as pl
from jax.experimental.pallas import tpu as pltpu
from jax.experimental.pallas import tpu_sc as plsc

POOL = 7
NPTS = POOL * POOL
NPAD = 56
LEVEL_W = (128, 64, 32, 16)
LEVEL_OFF = (0, 16384, 20480, 21504)
TABLE_ROWS = 21760


def _conv2d(x, W, b, pad):
    out = jax.lax.conv_general_dilated(
        x, W, (1, 1), [(pad, pad), (pad, pad)],
        dimension_numbers=('NCHW', 'OIHW', 'NCHW'))
    return out + b[None, :, None, None]


def _upsample_add(x, y):
    B, C, H, W = y.shape
    return jax.image.resize(x, (x.shape[0], x.shape[1], H, W), method='bilinear') + y


def _roi_meta(rois, im_info):
    R = rois.shape[0]
    h = rois[:, 4] - rois[:, 2] + 1.0
    w = rois[:, 3] - rois[:, 1] + 1.0
    lvl = jnp.floor(jnp.log(jnp.sqrt(h * w) / 224.0) / np.log(2.0) + 4.0)
    lvl = jnp.clip(lvl, 2.0, 5.0)
    li = lvl.astype(jnp.int32) - 2
    Wf = jnp.array(LEVEL_W, jnp.float32)[li]
    Wi = jnp.array(LEVEL_W, jnp.int32)[li]
    off = jnp.array(LEVEL_OFF, jnp.int32)[li]
    scale = Wf / im_info[0, 0]
    x1 = rois[:, 1] * scale
    y1 = rois[:, 2] * scale
    x2 = rois[:, 3] * scale
    y2 = rois[:, 4] * scale
    bw = jnp.maximum(x2 - x1 + 1.0, 1.0) / POOL
    bh = jnp.maximum(y2 - y1 + 1.0, 1.0) / POOL
    jj = jnp.arange(POOL, dtype=jnp.float32) + 0.5
    xs = x1[:, None] + jj[None, :] * bw[:, None]
    ys = y1[:, None] + jj[None, :] * bh[:, None]
    x0f = jnp.floor(xs)
    y0f = jnp.floor(ys)
    lx = xs - x0f
    ly = ys - y0f
    wmax = Wi[:, None] - 1
    x0 = jnp.clip(x0f.astype(jnp.int32), 0, wmax)
    x1i = jnp.clip(x0 + 1, 0, wmax)
    y0 = jnp.clip(y0f.astype(jnp.int32), 0, wmax)
    y1i = jnp.clip(y0 + 1, 0, wmax)
    ry0 = off[:, None] + y0 * Wi[:, None]
    ry1 = off[:, None] + y1i * Wi[:, None]

    def mk(rowy, xcol):
        return (rowy[:, :, None] + xcol[:, None, :]).reshape(R, NPTS)

    def mw(a, b):
        return (a[:, :, None] * b[:, None, :]).reshape(R, NPTS)

    idx = jnp.stack([mk(ry0, x0), mk(ry0, x1i), mk(ry1, x0), mk(ry1, x1i)], 1)
    wts = jnp.stack([mw(1.0 - ly, 1.0 - lx), mw(1.0 - ly, lx),
                     mw(ly, 1.0 - lx), mw(ly, lx)], 1)
    pad_rows = jnp.broadcast_to(jnp.arange(R, dtype=jnp.int32)[:, None, None],
                                (R, 4, NPAD - NPTS))
    idx = jnp.concatenate([idx.astype(jnp.int32), pad_rows], axis=2)
    wts = jnp.pad(wts, ((0, 0), (0, 0), (0, NPAD - NPTS)))
    return idx, wts


def _sc_gather(table, idx):
    R = idx.shape[0]
    rpw = R // 32
    mesh = plsc.VectorSubcoreMesh(core_axis_name="c", subcore_axis_name="s")

    @functools.partial(
        pl.kernel, mesh=mesh,
        out_type=jax.ShapeDtypeStruct((R, 2, 2 * NPAD, 256), jnp.float32),
        scratch_types=[
            pltpu.VMEM((rpw, 2, 2 * NPAD), jnp.int32),
            pltpu.VMEM((2, 2, 2 * NPAD, 256), jnp.float32),
            pltpu.SemaphoreType.DMA,
            pltpu.SemaphoreType.DMA,
            pltpu.SemaphoreType.DMA,
            pltpu.SemaphoreType.DMA,
        ],
    )
    def k(table_hbm, idx_hbm, out_hbm, idx_v, rows_v, gsem0, gsem1, ssem0, ssem1):
        wid = lax.axis_index("s") * 2 + lax.axis_index("c")
        base = wid * rpw
        pltpu.sync_copy(idx_hbm.at[pl.ds(base, rpw)], idx_v)
        gsem = (gsem0, gsem1)
        ssem = (ssem0, ssem1)
        pend_g = [None, None]
        pend_s = [None, None]

        def fire(i):
            s = i % 2
            pend_g[s] = [
                pltpu.async_copy(table_hbm.at[idx_v.at[i, j]],
                                 rows_v.at[s, j], gsem[s])
                for j in range(2)
            ]

        fire(0)
        for i in range(rpw):
            s = i % 2
            if i + 1 < rpw:
                if pend_s[1 - s] is not None:
                    pend_s[1 - s].wait()
                fire(i + 1)
            for h in pend_g[s]:
                h.wait()
            pend_s[s] = pltpu.async_copy(rows_v.at[s], out_hbm.at[base + i],
                                         ssem[s])
        pend_s[0].wait()
        pend_s[1].wait()

    return k(table, idx)


def _combine_body(w_ref, g_ref, o_ref):
    acc = (g_ref[:, 0] * w_ref[:, 0] + g_ref[:, 1] * w_ref[:, 1]
           + g_ref[:, 2] * w_ref[:, 2] + g_ref[:, 3] * w_ref[:, 3])
    o_ref[...] = acc[:, :NPTS, :]


def _tc_combine(g, wts):
    R = g.shape[0]
    BR = 8
    return pl.pallas_call(
        _combine_body,
        grid=(R // BR,),
        in_specs=[pl.BlockSpec((BR, 4, NPAD, 1), lambda i: (i, 0, 0, 0)),
                  pl.BlockSpec((BR, 4, NPAD, 256), lambda i: (i, 0, 0, 0))],
        out_specs=pl.BlockSpec((BR, NPTS, 256), lambda i: (i, 0, 0)),
        out_shape=jax.ShapeDtypeStruct((R, NPTS, 256), jnp.float32),
    )(wts[..., None], g)


def kernel(c2, c3, c4, c5, rois, im_info, Wt, bt, Wl1, bl1, Wl2, bl2, Wl3, bl3,
           Ws1, bs1, Ws2, bs2, Ws3, bs3):
    p5 = _conv2d(c5, Wt, bt, 0)
    p4 = _conv2d(_upsample_add(p5, _conv2d(c4, Wl1, bl1, 0)), Ws1, bs1, 1)
    p3 = _conv2d(_upsample_add(p4, _conv2d(c3, Wl2, bl2, 0)), Ws2, bs2, 1)
    p2 = _conv2d(_upsample_add(p3, _conv2d(c2, Wl3, bl3, 0)), Ws3, bs3, 1)

    C = Wt.shape[0]
    table = jnp.concatenate(
        [jnp.transpose(p, (0, 2, 3, 1)).reshape(-1, C) for p in (p2, p3, p4, p5)], 0)

    R = rois.shape[0]
    idx, wts = _roi_meta(rois, im_info)
    g = _sc_gather(table, idx.reshape(R, 2, 2 * NPAD))
    out = _tc_combine(g.reshape(R, 4, NPAD, C), wts)
    return jnp.transpose(out.reshape(R, POOL, POOL, C), (0, 3, 1, 2))
